# Initial kernel scaffold; baseline (speedup 1.0000x reference)
import jax
import jax.numpy as jnp
from jax.experimental import pallas as pl


def kernel(write_key, prev_read_weights, prev_usage_weights, alpha, mem):
    raise NotImplementedError("write your pallas kernel here")


# speedup vs baseline: 1.1977x; 1.1977x over previous
import functools

import jax
import jax.numpy as jnp
from jax import lax
from jax.experimental import pallas as pl

B, N, W = 256, 2048, 128
BB = 8


def _fused_body(wk_ref, read0_ref, usage_ref, alpha_ref, mem_ref,
                ww_ref, new_mem_ref):
    u = usage_ref[...]
    iota = lax.broadcasted_iota(jnp.int32, (BB, N), 1)
    minval = jnp.min(u, axis=1, keepdims=True)
    idx = jnp.min(jnp.where(u == minval, iota, N), axis=1, keepdims=True)
    onehot = (iota == idx).astype(jnp.float32)
    sig = jax.nn.sigmoid(alpha_ref[...])
    ww = sig * read0_ref[...] + (1.0 - sig) * onehot
    ww_ref[...] = ww
    wk = wk_ref[...]
    new_mem_ref[...] = mem_ref[...] + ww[:, :, None] * wk[:, None, :]


@jax.jit
def kernel(write_key, prev_read_weights, prev_usage_weights, alpha, mem):
    read0 = prev_read_weights[:, 0, :]
    grid = (B // BB,)
    ww, new_mem = pl.pallas_call(
        _fused_body,
        grid=grid,
        in_specs=[
            pl.BlockSpec((BB, W), lambda i: (i, 0)),
            pl.BlockSpec((BB, N), lambda i: (i, 0)),
            pl.BlockSpec((BB, N), lambda i: (i, 0)),
            pl.BlockSpec((BB, 1), lambda i: (i, 0)),
            pl.BlockSpec((BB, N, W), lambda i: (i, 0, 0)),
        ],
        out_specs=[
            pl.BlockSpec((BB, N), lambda i: (i, 0)),
            pl.BlockSpec((BB, N, W), lambda i: (i, 0, 0)),
        ],
        out_shape=[
            jax.ShapeDtypeStruct((B, N), jnp.float32),
            jax.ShapeDtypeStruct((B, N, W), jnp.float32),
        ],
    )(write_key, read0, prev_usage_weights, alpha, mem)
    return ww, new_mem
